# Initial kernel scaffold; baseline (speedup 1.0000x reference)
import jax
import jax.numpy as jnp
from jax.experimental import pallas as pl


def kernel(x1, edge_index1, batch1, x2, edge_index2, batch2, same_label, W1a, b1a, W1b, b1b, W2a, b2a, W2b, b2b, Wi, Wh, bi, bh, Wp, gamma, beta):
    raise NotImplementedError("write your pallas kernel here")


# speedup vs baseline: 3.3284x; 3.3284x over previous
import functools

import jax
import jax.numpy as jnp
from jax import lax
from jax.experimental import pallas as pl
from jax.experimental.pallas import tpu as pltpu
from jax.experimental.pallas import tpu_sc as plsc

N = 10000
E = 160000
D = 256
B = 64
STEPS = 3

_HALF = D // 2
_NSUB = 16
_EDGES_PER_TILE = E // _NSUB
_CH = 80
_NCHUNK = _EDGES_PER_TILE // _CH
_NPAD = 10240
_ROWS_PER_TILE = _NPAD // _NSUB


def _sc_scatter_add(x, src, dst, zeros_tile):
    table = x.reshape(2 * N, _HALF)
    mesh = plsc.VectorSubcoreMesh(core_axis_name="c", subcore_axis_name="s")

    @functools.partial(
        pl.kernel,
        mesh=mesh,
        out_type=jax.ShapeDtypeStruct((2, _NPAD, _HALF), jnp.float32),
        scratch_types=[
            pltpu.VMEM((_CH,), jnp.int32),
            pltpu.VMEM((_CH,), jnp.int32),
            pltpu.VMEM((_CH,), jnp.int32),
            pltpu.VMEM((_CH, _HALF), jnp.float32),
            pltpu.VMEM_SHARED((_NPAD, _HALF), jnp.float32),
            pltpu.SemaphoreType.DMA,
        ],
    )
    def k(table_hbm, src_hbm, dst_hbm, z_hbm, out_hbm,
          src_v, dst_v, idx_v, rows_v, acc_sh, sem):
        c = lax.axis_index("c")
        s = lax.axis_index("s")
        pltpu.sync_copy(z_hbm, acc_sh.at[pl.ds(s * _ROWS_PER_TILE, _ROWS_PER_TILE)])
        plsc.subcore_barrier()

        base = s * _EDGES_PER_TILE

        def body(i, carry):
            off = base + i * _CH
            pltpu.sync_copy(src_hbm.at[pl.ds(off, _CH)], src_v)
            pltpu.sync_copy(dst_hbm.at[pl.ds(off, _CH)], dst_v)
            for j in range(_CH // 16):
                sl = pl.ds(j * 16, 16)
                idx_v[sl] = src_v[sl] * 2 + c
            pltpu.async_copy(table_hbm.at[idx_v], rows_v, sem).wait()
            pltpu.sync_copy(rows_v, acc_sh.at[dst_v], add=True)
            return carry

        lax.fori_loop(0, _NCHUNK, body, 0)
        plsc.subcore_barrier()
        pltpu.sync_copy(
            acc_sh.at[pl.ds(s * _ROWS_PER_TILE, _ROWS_PER_TILE)],
            out_hbm.at[c, pl.ds(s * _ROWS_PER_TILE, _ROWS_PER_TILE)],
        )

    return k(table, src, dst, zeros_tile)


def _gin_mlp_tc(x, agg2, Wa, ba, Wb, bb):
    R = 1000
    dn = (((1,), (1,)), ((), ()))

    def body(x_ref, a0_ref, a1_ref, wa_ref, ba_ref, wb_ref, bb_ref, o_ref):
        h = x_ref[...] + jnp.concatenate([a0_ref[0], a1_ref[0]], axis=1)
        t = jax.lax.dot_general(h, wa_ref[...], dn,
                                preferred_element_type=jnp.float32)
        t = jnp.maximum(t + ba_ref[...], 0.0)
        o = jax.lax.dot_general(t, wb_ref[...], dn,
                                preferred_element_type=jnp.float32)
        o_ref[...] = jnp.maximum(o + bb_ref[...], 0.0)

    return pl.pallas_call(
        body,
        grid=(N // R,),
        in_specs=[
            pl.BlockSpec((R, D), lambda i: (i, 0)),
            pl.BlockSpec((1, R, _HALF), lambda i: (0, i, 0)),
            pl.BlockSpec((1, R, _HALF), lambda i: (1, i, 0)),
            pl.BlockSpec((D, D), lambda i: (0, 0)),
            pl.BlockSpec((1, D), lambda i: (0, 0)),
            pl.BlockSpec((D, D), lambda i: (0, 0)),
            pl.BlockSpec((1, D), lambda i: (0, 0)),
        ],
        out_specs=pl.BlockSpec((R, D), lambda i: (i, 0)),
        out_shape=jax.ShapeDtypeStruct((N, D), jnp.float32),
    )(x, agg2, agg2, Wa, ba, Wb, bb)


def _set2set_head_tc(h, batch_col, Wi, Wh, bi, bh, Wp, gamma, beta):
    dn11 = (((1,), (1,)), ((), ()))
    dn10 = (((1,), (0,)), ((), ()))
    dn00 = (((0,), (0,)), ((), ()))

    def _dot(u, v, dn):
        return jax.lax.dot_general(u, v, dn,
                                   preferred_element_type=jnp.float32)

    def _split(u):
        hi = u.astype(jnp.bfloat16)
        lo = (u - hi.astype(jnp.float32)).astype(jnp.bfloat16)
        return hi, lo

    def body(x_ref, b_ref, wi_ref, wh_ref, bi_ref, bh_ref, wp_ref,
             g_ref, be_ref, o_ref):
        x = x_ref[...]
        bcol = b_ref[...]
        iot = jax.lax.broadcasted_iota(jnp.int32, (N, B), 1)
        Mk = bcol == iot
        Mb = Mk.astype(jnp.bfloat16)
        x_hi, x_lo = _split(x)

        hs = jnp.zeros((B, D), jnp.float32)
        cs = jnp.zeros((B, D), jnp.float32)
        q = jnp.zeros((B, 2 * D), jnp.float32)
        for _ in range(STEPS):
            gates = (jax.lax.dot_general(q, wi_ref[...], dn11,
                                         preferred_element_type=jnp.float32)
                     + bi_ref[...]
                     + jax.lax.dot_general(hs, wh_ref[...], dn11,
                                           preferred_element_type=jnp.float32)
                     + bh_ref[...])
            ig = jax.nn.sigmoid(gates[:, :D])
            fg = jax.nn.sigmoid(gates[:, D:2 * D])
            gg = jnp.tanh(gates[:, 2 * D:3 * D])
            og = jax.nn.sigmoid(gates[:, 3 * D:])
            cs = fg * cs + ig * gg
            hs = og * jnp.tanh(cs)

            hs_hi, hs_lo = _split(hs)
            hb = _dot(Mb, hs_hi, dn10) + _dot(Mb, hs_lo, dn10)
            e = jnp.sum(x * hb, axis=1, keepdims=True)
            S = jnp.where(Mk, e, -1e30)
            emax = jnp.max(S, axis=0, keepdims=True)
            e_pn = jnp.max(jnp.where(Mk, emax, -1e30), axis=1, keepdims=True)
            ee = jnp.exp(e - e_pn)
            den = jnp.sum(jnp.where(Mk, ee, 0.0), axis=0, keepdims=True)
            den_pn = jnp.max(jnp.where(Mk, den, 0.0), axis=1, keepdims=True)
            a = ee / (den_pn + 1e-16)
            Ma = jnp.where(Mk, a, 0.0)
            Ma_hi, Ma_lo = _split(Ma)
            r = (_dot(Ma_hi, x_hi, dn00) + _dot(Ma_hi, x_lo, dn00)
                 + _dot(Ma_lo, x_hi, dn00))
            q = jnp.concatenate([hs, r], axis=1)

        z = jax.lax.dot_general(q, wp_ref[...], dn11,
                                preferred_element_type=jnp.float32)
        mu = jnp.mean(z, axis=0, keepdims=True)
        var = jnp.mean((z - mu) ** 2, axis=0, keepdims=True)
        zn = (z - mu) / jnp.sqrt(var + 1e-5) * g_ref[...] + be_ref[...]
        nrm = jnp.maximum(jnp.sqrt(jnp.sum(zn * zn, axis=1, keepdims=True)),
                          1e-12)
        o_ref[...] = zn / nrm

    return pl.pallas_call(
        body,
        grid=(1,),
        in_specs=[
            pl.BlockSpec((N, D), lambda i: (0, 0)),
            pl.BlockSpec((N, 1), lambda i: (0, 0)),
            pl.BlockSpec((4 * D, 2 * D), lambda i: (0, 0)),
            pl.BlockSpec((4 * D, D), lambda i: (0, 0)),
            pl.BlockSpec((1, 4 * D), lambda i: (0, 0)),
            pl.BlockSpec((1, 4 * D), lambda i: (0, 0)),
            pl.BlockSpec((2 * D, 2 * D), lambda i: (0, 0)),
            pl.BlockSpec((1, 2 * D), lambda i: (0, 0)),
            pl.BlockSpec((1, 2 * D), lambda i: (0, 0)),
        ],
        out_specs=pl.BlockSpec((B, 2 * D), lambda i: (0, 0)),
        out_shape=jax.ShapeDtypeStruct((B, 2 * D), jnp.float32),
    )(h, batch_col, Wi, Wh, bi, bh, Wp, gamma, beta)


def _encode(x, edge_index, batch, zeros_tile, p):
    src = edge_index[0]
    dst = edge_index[1]
    agg1 = _sc_scatter_add(x, src, dst, zeros_tile)
    h1 = _gin_mlp_tc(x, agg1, p['W1a'], p['b1a'], p['W1b'], p['b1b'])
    agg2 = _sc_scatter_add(h1, src, dst, zeros_tile)
    h2 = _gin_mlp_tc(h1, agg2, p['W2a'], p['b2a'], p['W2b'], p['b2b'])
    return _set2set_head_tc(h2, batch[:, None], p['Wi'], p['Wh'],
                            p['bi'], p['bh'], p['Wp'], p['gamma'], p['beta'])


def kernel(x1, edge_index1, batch1, x2, edge_index2, batch2, same_label,
           W1a, b1a, W1b, b1b, W2a, b2a, W2b, b2b, Wi, Wh, bi, bh,
           Wp, gamma, beta):
    p = dict(
        W1a=W1a, b1a=b1a.reshape(1, D), W1b=W1b, b1b=b1b.reshape(1, D),
        W2a=W2a, b2a=b2a.reshape(1, D), W2b=W2b, b2b=b2b.reshape(1, D),
        Wi=Wi, Wh=Wh, bi=bi.reshape(1, 4 * D), bh=bh.reshape(1, 4 * D),
        Wp=Wp, gamma=gamma.reshape(1, 2 * D), beta=beta.reshape(1, 2 * D),
    )
    zeros_tile = jnp.zeros((_ROWS_PER_TILE, _HALF), jnp.float32)
    z1 = _encode(x1, edge_index1, batch1, zeros_tile, p)
    z2 = _encode(x2, edge_index2, batch2, zeros_tile, p)
    return (z1, z2)
